# baseline (speedup 1.0000x reference)
import functools

import jax
import jax.numpy as jnp
from jax import lax
from jax.experimental import pallas as pl
from jax.experimental.pallas import tpu as pltpu
from jax.experimental.pallas import tpu_sc as plsc

N = 10000
H = 256
G = 256
OUT = 128
NROWS = 10240
CH = 128
SUPE = 40
EP = 16 * SUPE * 2 * CH
SUPM = 24
EPM = 16 * SUPM * 2 * CH
SUPD = 10
TROWS = 9 * 119

_f32 = jnp.float32
_mesh = plsc.VectorSubcoreMesh(core_axis_name="c", subcore_axis_name="s")
_sc_params = pltpu.CompilerParams(use_tc_tiling_on_sc=False)


def _fill2d(ref, nrows, value):
    def body(r, carry):
        ref[r, pl.ds(0, 16)] = jnp.full((16,), value, _f32)
        return carry
    lax.fori_loop(0, nrows, body, 0)


def _zero_buf(ref):
    def body(r, carry):
        for c in range(8):
            ref[r, pl.ds(c * 16, 16)] = jnp.zeros((16,), _f32)
        return carry
    lax.fori_loop(0, CH, body, 0)


def _gather_scatter_loop(table, srch, dsth, idxb, dstb, bufs, acc,
                         semg, sems, cid, sid, nsup):
    wrow = (cid * 16 + sid) * nsup
    drow = sid * nsup

    def super_body(s, carry):
        pltpu.sync_copy(srch.at[wrow + s], idxb)
        g0 = pltpu.async_copy(table.at[idxb.at[0]], bufs.at[0], semg)
        g1 = pltpu.async_copy(table.at[idxb.at[1]], bufs.at[1], semg)
        pltpu.sync_copy(dsth.at[drow + s], dstb)
        g0.wait()
        s0 = pltpu.async_copy(bufs.at[0], acc.at[dstb.at[0]], sems, add=True)
        g1.wait()
        s1 = pltpu.async_copy(bufs.at[1], acc.at[dstb.at[1]], sems, add=True)
        s0.wait()
        s1.wait()
        return carry
    lax.fori_loop(0, nsup, super_body, 0)


def _zero_acc(bufs, acc, sid):
    _zero_buf(bufs.at[0])
    for j in range(5):
        pltpu.sync_copy(bufs.at[0], acc.at[pl.ds(sid * 640 + j * CH, CH)])


def _writeout(acc, out, cid, sid):
    for j in range(5):
        pltpu.sync_copy(acc.at[pl.ds(sid * 640 + j * CH, CH)],
                        out.at[pl.ds(cid * NROWS + sid * 640 + j * CH, CH)])


@functools.partial(
    pl.kernel,
    out_type=jax.ShapeDtypeStruct((2 * NROWS, 128), _f32),
    mesh=_mesh,
    compiler_params=_sc_params,
    scratch_types=[
        pltpu.VMEM((2, CH), jnp.int32),
        pltpu.VMEM((2, CH), jnp.int32),
        pltpu.VMEM((2, CH, 128), _f32),
        pltpu.VMEM_SHARED((NROWS, 128), _f32),
        pltpu.SemaphoreType.DMA,
        pltpu.SemaphoreType.DMA,
    ],
)
def _sc_emb(tcol, esrc2, edst, x0_out, idxb, dstb, bufs, acc, semg, sems):
    cid = lax.axis_index("c")
    sid = lax.axis_index("s")
    _zero_acc(bufs, acc, sid)
    plsc.subcore_barrier()
    _gather_scatter_loop(tcol, esrc2, edst, idxb, dstb, bufs, acc,
                         semg, sems, cid, sid, SUPM)
    plsc.subcore_barrier()
    _writeout(acc, x0_out, cid, sid)


@functools.partial(
    pl.kernel,
    out_type=jax.ShapeDtypeStruct((2 * NROWS, 16), _f32),
    mesh=_mesh,
    compiler_params=_sc_params,
    scratch_types=[
        pltpu.VMEM((8, CH), jnp.int32),
        pltpu.VMEM((CH, 16), _f32),
        pltpu.VMEM((640, 16), _f32),
        pltpu.VMEM_SHARED((NROWS, 16), _f32),
        pltpu.SemaphoreType.DMA,
    ],
)
def _sc_deg(dste, dstc, deg_out, dstb8, ones16, z16, accdeg, semd):
    cid = lax.axis_index("c")
    sid = lax.axis_index("s")
    _fill2d(z16, 640, 0.0)
    _fill2d(ones16, CH, 1.0)
    pltpu.sync_copy(z16, accdeg.at[pl.ds(sid * 640, 640)])
    plsc.subcore_barrier()

    def deg_super(s, carry):
        @pl.when(cid == 0)
        def _():
            pltpu.sync_copy(dste.at[sid * SUPD + s], dstb8)

        @pl.when(cid == 1)
        def _():
            pltpu.sync_copy(dstc.at[sid * SUPD + s], dstb8)

        hs = [pltpu.async_copy(ones16, accdeg.at[dstb8.at[b]], semd, add=True)
              for b in range(8)]
        for h in hs:
            h.wait()
        return carry
    lax.fori_loop(0, SUPD, deg_super, 0)

    plsc.subcore_barrier()
    pltpu.sync_copy(accdeg.at[pl.ds(sid * 640, 640)],
                    deg_out.at[pl.ds(cid * NROWS + sid * 640, 640)])


@functools.partial(
    pl.kernel,
    out_type=jax.ShapeDtypeStruct((2 * NROWS, 128), _f32),
    mesh=_mesh,
    compiler_params=_sc_params,
    scratch_types=[
        pltpu.VMEM((2, CH), jnp.int32),
        pltpu.VMEM((2, CH), jnp.int32),
        pltpu.VMEM((2, CH, 128), _f32),
        pltpu.VMEM_SHARED((NROWS, 128), _f32),
        pltpu.SemaphoreType.DMA,
        pltpu.SemaphoreType.DMA,
    ],
)
def _sc_agg(table, src2, dstp, out, idxb, dstb, bufs, acc, semg, sems):
    cid = lax.axis_index("c")
    sid = lax.axis_index("s")
    _zero_acc(bufs, acc, sid)
    plsc.subcore_barrier()
    _gather_scatter_loop(table, src2, dstp, idxb, dstb, bufs, acc,
                         semg, sems, cid, sid, SUPE)
    plsc.subcore_barrier()
    _writeout(acc, out, cid, sid)


_B = 1000


def _k1_body(x0l, x0r, he, hc, W0, Wc, h0_o, de_o, dc_o, hws0_o, hwsc_o):
    h = jnp.concatenate([x0l[...], x0r[...]], axis=1)
    h0_o[...] = h
    de = lax.rsqrt(he[...] + 1.0)
    dc = lax.rsqrt(hc[...] + 1.0)
    de_o[...] = de
    dc_o[...] = dc
    hws0_o[...] = jnp.dot(h, W0[...], preferred_element_type=_f32) * de
    hwsc_o[...] = jnp.dot(h, Wc[...], preferred_element_type=_f32) * dc


def _tc_prep(x0l, x0r, hist_e, hist_c, W0, Wc):
    return pl.pallas_call(
        _k1_body,
        grid=(10, 2),
        in_specs=[
            pl.BlockSpec((_B, 128), lambda i, j: (i, 0)),
            pl.BlockSpec((_B, 128), lambda i, j: (i, 0)),
            pl.BlockSpec((_B, 1), lambda i, j: (i, 0)),
            pl.BlockSpec((_B, 1), lambda i, j: (i, 0)),
            pl.BlockSpec((H, 128), lambda i, j: (0, j)),
            pl.BlockSpec((H, 128), lambda i, j: (0, j)),
        ],
        out_specs=[
            pl.BlockSpec((_B, H), lambda i, j: (i, 0)),
            pl.BlockSpec((_B, 1), lambda i, j: (i, 0)),
            pl.BlockSpec((_B, 1), lambda i, j: (i, 0)),
            pl.BlockSpec((_B, 128), lambda i, j: (10 * j + i, 0)),
            pl.BlockSpec((_B, 128), lambda i, j: (10 * j + i, 0)),
        ],
        out_shape=[
            jax.ShapeDtypeStruct((N, H), _f32),
            jax.ShapeDtypeStruct((N, 1), _f32),
            jax.ShapeDtypeStruct((N, 1), _f32),
            jax.ShapeDtypeStruct((2 * N, 128), _f32),
            jax.ShapeDtypeStruct((2 * N, 128), _f32),
        ],
    )(x0l, x0r, hist_e, hist_c, W0, Wc)


def _k2_body(al, ar, hl, hr, dv, b, y_o, s_o, q_o):
    i = pl.program_id(0)
    agg = jnp.concatenate([al[...], ar[...]], axis=1)
    hws = jnp.concatenate([hl[...], hr[...]], axis=1)
    y = dv[...] * (agg + hws) + b[...]
    y_o[...] = y

    @pl.when(i == 0)
    def _():
        s_o[...] = jnp.zeros_like(s_o)
        q_o[...] = jnp.zeros_like(q_o)

    s_o[...] += jnp.sum(y, axis=0, keepdims=True)
    q_o[...] += jnp.sum(y * y, axis=0, keepdims=True)


def _tc_stats(al, ar, hl, hr, dv, b):
    return pl.pallas_call(
        _k2_body,
        grid=(10,),
        in_specs=[
            pl.BlockSpec((_B, 128), lambda i: (i, 0)),
            pl.BlockSpec((_B, 128), lambda i: (i, 0)),
            pl.BlockSpec((_B, 128), lambda i: (i, 0)),
            pl.BlockSpec((_B, 128), lambda i: (i, 0)),
            pl.BlockSpec((_B, 1), lambda i: (i, 0)),
            pl.BlockSpec((1, H), lambda i: (0, 0)),
        ],
        out_specs=[
            pl.BlockSpec((_B, H), lambda i: (i, 0)),
            pl.BlockSpec((1, H), lambda i: (0, 0)),
            pl.BlockSpec((1, H), lambda i: (0, 0)),
        ],
        out_shape=[
            jax.ShapeDtypeStruct((N, H), _f32),
            jax.ShapeDtypeStruct((1, H), _f32),
            jax.ShapeDtypeStruct((1, H), _f32),
        ],
    )(al, ar, hl, hr, dv, b)


def _k3_body(y, s, q, g, be, hp, Wn, dv, hn_o, hws_o):
    mu = s[...] / N
    var = q[...] / N - mu * mu
    sc = g[...] * lax.rsqrt(var + 1e-5)
    sh = be[...] - mu * sc
    hn = jnp.maximum(y[...] * sc + sh, 0.0) + hp[...]
    hn_o[...] = hn
    hws_o[...] = jnp.dot(hn, Wn[...], preferred_element_type=_f32) * dv[...]


def _tc_norm_next(y, s, q, g, be, hp, Wn, dv):
    return pl.pallas_call(
        _k3_body,
        grid=(10, 2),
        in_specs=[
            pl.BlockSpec((_B, H), lambda i, j: (i, 0)),
            pl.BlockSpec((1, H), lambda i, j: (0, 0)),
            pl.BlockSpec((1, H), lambda i, j: (0, 0)),
            pl.BlockSpec((1, H), lambda i, j: (0, 0)),
            pl.BlockSpec((1, H), lambda i, j: (0, 0)),
            pl.BlockSpec((_B, H), lambda i, j: (i, 0)),
            pl.BlockSpec((H, 128), lambda i, j: (0, j)),
            pl.BlockSpec((_B, 1), lambda i, j: (i, 0)),
        ],
        out_specs=[
            pl.BlockSpec((_B, H), lambda i, j: (i, 0)),
            pl.BlockSpec((_B, 128), lambda i, j: (10 * j + i, 0)),
        ],
        out_shape=[
            jax.ShapeDtypeStruct((N, H), _f32),
            jax.ShapeDtypeStruct((2 * N, 128), _f32),
        ],
    )(y, s, q, g, be, hp, Wn, dv)


def _k4_body(y, s, q, g, be, hp, x_o):
    mu = s[...] / N
    var = q[...] / N - mu * mu
    sc = g[...] * lax.rsqrt(var + 1e-5)
    sh = be[...] - mu * sc
    x_o[...] = jnp.maximum(y[...] * sc + sh, 0.0) + hp[...]


def _tc_norm_final(y, s, q, g, be, hp):
    return pl.pallas_call(
        _k4_body,
        grid=(10,),
        in_specs=[
            pl.BlockSpec((_B, H), lambda i: (i, 0)),
            pl.BlockSpec((1, H), lambda i: (0, 0)),
            pl.BlockSpec((1, H), lambda i: (0, 0)),
            pl.BlockSpec((1, H), lambda i: (0, 0)),
            pl.BlockSpec((1, H), lambda i: (0, 0)),
            pl.BlockSpec((_B, H), lambda i: (i, 0)),
        ],
        out_specs=pl.BlockSpec((_B, H), lambda i: (i, 0)),
        out_shape=jax.ShapeDtypeStruct((N, H), _f32),
    )(y, s, q, g, be, hp)


def _k5_body(xa, xb, bt, linW, linb, out_o, Pa, Pb, C8):
    i = pl.program_id(0)

    @pl.when(i == 0)
    def _():
        Pa[...] = jnp.zeros_like(Pa)
        Pb[...] = jnp.zeros_like(Pb)
        C8[...] = jnp.zeros_like(C8)

    oh = (bt[...] == lax.broadcasted_iota(jnp.int32, (1, G), 1)).astype(_f32)
    dims = (((0,), (0,)), ((), ()))
    Pa[...] += lax.dot_general(oh, xa[...], dims, preferred_element_type=_f32)
    Pb[...] += lax.dot_general(oh, xb[...], dims, preferred_element_type=_f32)
    C8[...] += lax.dot_general(oh, jnp.ones((_B, 8), _f32), dims,
                               preferred_element_type=_f32)

    @pl.when(i == 9)
    def _():
        cnt = jnp.maximum(C8[...][:, 0:1], 1.0)
        z = jnp.concatenate([Pa[...] / cnt, Pb[...] / cnt], axis=1)
        out_o[...] = jnp.dot(z, linW[...], preferred_element_type=_f32) + linb[...]


def _tc_pool(xa, xb, bt, linW, linb):
    return pl.pallas_call(
        _k5_body,
        grid=(10,),
        in_specs=[
            pl.BlockSpec((_B, H), lambda i: (i, 0)),
            pl.BlockSpec((_B, H), lambda i: (i, 0)),
            pl.BlockSpec((_B, 1), lambda i: (i, 0)),
            pl.BlockSpec((2 * H, OUT), lambda i: (0, 0)),
            pl.BlockSpec((1, OUT), lambda i: (0, 0)),
        ],
        out_specs=pl.BlockSpec((G, OUT), lambda i: (0, 0)),
        out_shape=jax.ShapeDtypeStruct((G, OUT), _f32),
        scratch_shapes=[
            pltpu.VMEM((G, H), _f32),
            pltpu.VMEM((G, H), _f32),
            pltpu.VMEM((G, 8), _f32),
        ],
    )(xa, xb, bt, linW, linb)


def _pad1(a, length, value):
    return jnp.concatenate(
        [a, jnp.full((length - a.shape[0],), value, a.dtype)])


def kernel(x, edge_index, cycle_index, batch, atom_tables, W0, b0, g0, be0,
           W1, b1, g1, be1, W2, b2, g2, be2, Wc, bc, gc, bec, lin_W, lin_b):
    i32 = jnp.int32
    idx_emb = (x.astype(i32).T + 119 * jnp.arange(9, dtype=i32)[:, None]).reshape(-1)
    node_rep = jnp.tile(jnp.arange(N, dtype=i32), 9)
    ep = _pad1(idx_emb, EPM, 0)
    emb_src2 = jnp.stack([ep, ep + TROWS]).reshape(32 * SUPM, 2, CH)
    emb_dst = _pad1(node_rep, EPM, N).reshape(16 * SUPM, 2, CH)

    def edge_prep(ei):
        src = ei[0].astype(i32)
        order = jnp.argsort(src)
        sp = _pad1(src[order], EP, 0)
        src2 = jnp.stack([sp, sp + N]).reshape(32 * SUPE, 2, CH)
        dp = _pad1(ei[1].astype(i32)[order], EP, N)
        dstp = dp.reshape(16 * SUPE, 2, CH)
        dstd = dp.reshape(16 * SUPD, 8, CH)
        return src2, dstp, dstd

    src2_e, dstp_e, dstd_e = edge_prep(edge_index)
    src2_c, dstp_c, dstd_c = edge_prep(cycle_index)
    tcol = atom_tables.reshape(TROWS, 2, 128).transpose(1, 0, 2).reshape(2 * TROWS, 128)

    x0c = _sc_emb(tcol, emb_src2, emb_dst)
    degc = _sc_deg(dstd_e, dstd_c)
    x0l, x0r = x0c[0:N], x0c[NROWS:NROWS + N]
    hist_e = degc[0:N, 0:1]
    hist_c = degc[NROWS:NROWS + N, 0:1]

    h0, dinv_e, dinv_c, hws0, hwsc = _tc_prep(x0l, x0r, hist_e, hist_c, W0, Wc)

    def agg(hws, src2, dstp):
        a = _sc_agg(hws, src2, dstp)
        return a[0:N], a[NROWS:NROWS + N]

    b0r, g0r, be0r = b0.reshape(1, H), g0.reshape(1, H), be0.reshape(1, H)
    b1r, g1r, be1r = b1.reshape(1, H), g1.reshape(1, H), be1.reshape(1, H)
    b2r, g2r, be2r = b2.reshape(1, H), g2.reshape(1, H), be2.reshape(1, H)
    bcr, gcr, becr = bc.reshape(1, H), gc.reshape(1, H), bec.reshape(1, H)

    a0l, a0r = agg(hws0, src2_e, dstp_e)
    y0, s0, q0 = _tc_stats(a0l, a0r, hws0[0:N], hws0[N:2 * N], dinv_e, b0r)
    h1, hws1 = _tc_norm_next(y0, s0, q0, g0r, be0r, h0, W1, dinv_e)
    a1l, a1r = agg(hws1, src2_e, dstp_e)
    y1, s1, q1 = _tc_stats(a1l, a1r, hws1[0:N], hws1[N:2 * N], dinv_e, b1r)
    h2, hws2 = _tc_norm_next(y1, s1, q1, g1r, be1r, h1, W2, dinv_e)
    a2l, a2r = agg(hws2, src2_e, dstp_e)
    y2, s2, q2 = _tc_stats(a2l, a2r, hws2[0:N], hws2[N:2 * N], dinv_e, b2r)
    x_out_a = _tc_norm_final(y2, s2, q2, g2r, be2r, h2)
    acl, acr = agg(hwsc, src2_c, dstp_c)
    yc, scs, qc = _tc_stats(acl, acr, hwsc[0:N], hwsc[N:2 * N], dinv_c, bcr)
    x_out_b = _tc_norm_final(yc, scs, qc, gcr, becr, h0)

    bt = batch.astype(i32).reshape(N, 1)
    return _tc_pool(x_out_a, x_out_b, bt, lin_W, lin_b.reshape(1, OUT))


# speedup vs baseline: 1.2882x; 1.2882x over previous
import functools

import jax
import jax.numpy as jnp
from jax import lax
from jax.experimental import pallas as pl
from jax.experimental.pallas import tpu as pltpu
from jax.experimental.pallas import tpu_sc as plsc

N = 10000
H = 256
G = 256
OUT = 128
NROWS = 10240
CH = 64
NB = 4
SUPE = 40
EP = 16 * SUPE * NB * CH
SUPM = 24
EPM = 16 * SUPM * NB * CH
SUPD = 10
DCH = 128
TROWS = 9 * 119

_f32 = jnp.float32
_mesh = plsc.VectorSubcoreMesh(core_axis_name="c", subcore_axis_name="s")
_sc_params = pltpu.CompilerParams(use_tc_tiling_on_sc=False)


def _fill2d(ref, nrows, value):
    def body(r, carry):
        ref[r, pl.ds(0, 16)] = jnp.full((16,), value, _f32)
        return carry
    lax.fori_loop(0, nrows, body, 0)


def _zero_buf(ref):
    def body(r, carry):
        for c in range(8):
            ref[r, pl.ds(c * 16, 16)] = jnp.zeros((16,), _f32)
        return carry
    lax.fori_loop(0, CH, body, 0)


def _gather_scatter_loop(table, srch, dsth, idxb, dstb, bufs, acc,
                         semg, sems, cid, sid, nsup):
    wrow = (cid * 16 + sid) * nsup
    drow = sid * nsup

    def super_body(s, carry):
        pltpu.sync_copy(srch.at[wrow + s], idxb)
        gs = [pltpu.async_copy(table.at[idxb.at[b]], bufs.at[b], semg)
              for b in range(NB)]
        pltpu.sync_copy(dsth.at[drow + s], dstb)
        ss = []
        for b in range(NB):
            gs[b].wait()
            ss.append(pltpu.async_copy(bufs.at[b], acc.at[dstb.at[b]], sems,
                                       add=True))
        for h in ss:
            h.wait()
        return carry
    lax.fori_loop(0, nsup, super_body, 0)


def _zero_acc(bufs, acc, sid):
    _zero_buf(bufs.at[0])
    for j in range(10):
        pltpu.sync_copy(bufs.at[0], acc.at[pl.ds(sid * 640 + j * CH, CH)])


def _writeout(acc, out, cid, sid):
    for j in range(10):
        pltpu.sync_copy(acc.at[pl.ds(sid * 640 + j * CH, CH)],
                        out.at[pl.ds(cid * NROWS + sid * 640 + j * CH, CH)])


@functools.partial(
    pl.kernel,
    out_type=jax.ShapeDtypeStruct((2 * NROWS, 128), _f32),
    mesh=_mesh,
    compiler_params=_sc_params,
    scratch_types=[
        pltpu.VMEM((NB, CH), jnp.int32),
        pltpu.VMEM((NB, CH), jnp.int32),
        pltpu.VMEM((NB, CH, 128), _f32),
        pltpu.VMEM_SHARED((NROWS, 128), _f32),
        pltpu.SemaphoreType.DMA,
        pltpu.SemaphoreType.DMA,
    ],
)
def _sc_emb(tcol, esrc2, edst, x0_out, idxb, dstb, bufs, acc, semg, sems):
    cid = lax.axis_index("c")
    sid = lax.axis_index("s")
    _zero_acc(bufs, acc, sid)
    plsc.subcore_barrier()
    _gather_scatter_loop(tcol, esrc2, edst, idxb, dstb, bufs, acc,
                         semg, sems, cid, sid, SUPM)
    plsc.subcore_barrier()
    _writeout(acc, x0_out, cid, sid)


@functools.partial(
    pl.kernel,
    out_type=jax.ShapeDtypeStruct((2 * NROWS, 16), _f32),
    mesh=_mesh,
    compiler_params=_sc_params,
    scratch_types=[
        pltpu.VMEM((8, DCH), jnp.int32),
        pltpu.VMEM((DCH, 16), _f32),
        pltpu.VMEM((640, 16), _f32),
        pltpu.VMEM_SHARED((NROWS, 16), _f32),
        pltpu.SemaphoreType.DMA,
    ],
)
def _sc_deg(dste, dstc, deg_out, dstb8, ones16, z16, accdeg, semd):
    cid = lax.axis_index("c")
    sid = lax.axis_index("s")
    _fill2d(z16, 640, 0.0)
    _fill2d(ones16, DCH, 1.0)
    pltpu.sync_copy(z16, accdeg.at[pl.ds(sid * 640, 640)])
    plsc.subcore_barrier()

    def deg_super(s, carry):
        @pl.when(cid == 0)
        def _():
            pltpu.sync_copy(dste.at[sid * SUPD + s], dstb8)

        @pl.when(cid == 1)
        def _():
            pltpu.sync_copy(dstc.at[sid * SUPD + s], dstb8)

        hs = [pltpu.async_copy(ones16, accdeg.at[dstb8.at[b]], semd, add=True)
              for b in range(8)]
        for h in hs:
            h.wait()
        return carry
    lax.fori_loop(0, SUPD, deg_super, 0)

    plsc.subcore_barrier()
    pltpu.sync_copy(accdeg.at[pl.ds(sid * 640, 640)],
                    deg_out.at[pl.ds(cid * NROWS + sid * 640, 640)])


@functools.partial(
    pl.kernel,
    out_type=jax.ShapeDtypeStruct((2 * NROWS, 128), _f32),
    mesh=_mesh,
    compiler_params=_sc_params,
    scratch_types=[
        pltpu.VMEM((NB, CH), jnp.int32),
        pltpu.VMEM((NB, CH), jnp.int32),
        pltpu.VMEM((NB, CH, 128), _f32),
        pltpu.VMEM_SHARED((NROWS, 128), _f32),
        pltpu.SemaphoreType.DMA,
        pltpu.SemaphoreType.DMA,
    ],
)
def _sc_agg(table, src2, dstp, out, idxb, dstb, bufs, acc, semg, sems):
    cid = lax.axis_index("c")
    sid = lax.axis_index("s")
    _zero_acc(bufs, acc, sid)
    plsc.subcore_barrier()
    _gather_scatter_loop(table, src2, dstp, idxb, dstb, bufs, acc,
                         semg, sems, cid, sid, SUPE)
    plsc.subcore_barrier()
    _writeout(acc, out, cid, sid)


_B = 1000


def _k1_body(x0l, x0r, he, hc, W0, Wc, h0_o, de_o, dc_o, hws0_o, hwsc_o):
    h = jnp.concatenate([x0l[...], x0r[...]], axis=1)
    h0_o[...] = h
    de = lax.rsqrt(he[...] + 1.0)
    dc = lax.rsqrt(hc[...] + 1.0)
    de_o[...] = de
    dc_o[...] = dc
    hws0_o[...] = jnp.dot(h, W0[...], preferred_element_type=_f32) * de
    hwsc_o[...] = jnp.dot(h, Wc[...], preferred_element_type=_f32) * dc


def _tc_prep(x0l, x0r, hist_e, hist_c, W0, Wc):
    return pl.pallas_call(
        _k1_body,
        grid=(10, 2),
        in_specs=[
            pl.BlockSpec((_B, 128), lambda i, j: (i, 0)),
            pl.BlockSpec((_B, 128), lambda i, j: (i, 0)),
            pl.BlockSpec((_B, 1), lambda i, j: (i, 0)),
            pl.BlockSpec((_B, 1), lambda i, j: (i, 0)),
            pl.BlockSpec((H, 128), lambda i, j: (0, j)),
            pl.BlockSpec((H, 128), lambda i, j: (0, j)),
        ],
        out_specs=[
            pl.BlockSpec((_B, H), lambda i, j: (i, 0)),
            pl.BlockSpec((_B, 1), lambda i, j: (i, 0)),
            pl.BlockSpec((_B, 1), lambda i, j: (i, 0)),
            pl.BlockSpec((_B, 128), lambda i, j: (10 * j + i, 0)),
            pl.BlockSpec((_B, 128), lambda i, j: (10 * j + i, 0)),
        ],
        out_shape=[
            jax.ShapeDtypeStruct((N, H), _f32),
            jax.ShapeDtypeStruct((N, 1), _f32),
            jax.ShapeDtypeStruct((N, 1), _f32),
            jax.ShapeDtypeStruct((2 * N, 128), _f32),
            jax.ShapeDtypeStruct((2 * N, 128), _f32),
        ],
    )(x0l, x0r, hist_e, hist_c, W0, Wc)


def _k2_body(al, ar, hl, hr, dv, b, y_o, s_o, q_o):
    i = pl.program_id(0)
    agg = jnp.concatenate([al[...], ar[...]], axis=1)
    hws = jnp.concatenate([hl[...], hr[...]], axis=1)
    y = dv[...] * (agg + hws) + b[...]
    y_o[...] = y

    @pl.when(i == 0)
    def _():
        s_o[...] = jnp.zeros_like(s_o)
        q_o[...] = jnp.zeros_like(q_o)

    s_o[...] += jnp.sum(y, axis=0, keepdims=True)
    q_o[...] += jnp.sum(y * y, axis=0, keepdims=True)


def _tc_stats(al, ar, hl, hr, dv, b):
    return pl.pallas_call(
        _k2_body,
        grid=(10,),
        in_specs=[
            pl.BlockSpec((_B, 128), lambda i: (i, 0)),
            pl.BlockSpec((_B, 128), lambda i: (i, 0)),
            pl.BlockSpec((_B, 128), lambda i: (i, 0)),
            pl.BlockSpec((_B, 128), lambda i: (i, 0)),
            pl.BlockSpec((_B, 1), lambda i: (i, 0)),
            pl.BlockSpec((1, H), lambda i: (0, 0)),
        ],
        out_specs=[
            pl.BlockSpec((_B, H), lambda i: (i, 0)),
            pl.BlockSpec((1, H), lambda i: (0, 0)),
            pl.BlockSpec((1, H), lambda i: (0, 0)),
        ],
        out_shape=[
            jax.ShapeDtypeStruct((N, H), _f32),
            jax.ShapeDtypeStruct((1, H), _f32),
            jax.ShapeDtypeStruct((1, H), _f32),
        ],
    )(al, ar, hl, hr, dv, b)


def _k3_body(y, s, q, g, be, hp, Wn, dv, hn_o, hws_o):
    mu = s[...] / N
    var = q[...] / N - mu * mu
    sc = g[...] * lax.rsqrt(var + 1e-5)
    sh = be[...] - mu * sc
    hn = jnp.maximum(y[...] * sc + sh, 0.0) + hp[...]
    hn_o[...] = hn
    hws_o[...] = jnp.dot(hn, Wn[...], preferred_element_type=_f32) * dv[...]


def _tc_norm_next(y, s, q, g, be, hp, Wn, dv):
    return pl.pallas_call(
        _k3_body,
        grid=(10, 2),
        in_specs=[
            pl.BlockSpec((_B, H), lambda i, j: (i, 0)),
            pl.BlockSpec((1, H), lambda i, j: (0, 0)),
            pl.BlockSpec((1, H), lambda i, j: (0, 0)),
            pl.BlockSpec((1, H), lambda i, j: (0, 0)),
            pl.BlockSpec((1, H), lambda i, j: (0, 0)),
            pl.BlockSpec((_B, H), lambda i, j: (i, 0)),
            pl.BlockSpec((H, 128), lambda i, j: (0, j)),
            pl.BlockSpec((_B, 1), lambda i, j: (i, 0)),
        ],
        out_specs=[
            pl.BlockSpec((_B, H), lambda i, j: (i, 0)),
            pl.BlockSpec((_B, 128), lambda i, j: (10 * j + i, 0)),
        ],
        out_shape=[
            jax.ShapeDtypeStruct((N, H), _f32),
            jax.ShapeDtypeStruct((2 * N, 128), _f32),
        ],
    )(y, s, q, g, be, hp, Wn, dv)


def _k4_body(y, s, q, g, be, hp, x_o):
    mu = s[...] / N
    var = q[...] / N - mu * mu
    sc = g[...] * lax.rsqrt(var + 1e-5)
    sh = be[...] - mu * sc
    x_o[...] = jnp.maximum(y[...] * sc + sh, 0.0) + hp[...]


def _tc_norm_final(y, s, q, g, be, hp):
    return pl.pallas_call(
        _k4_body,
        grid=(10,),
        in_specs=[
            pl.BlockSpec((_B, H), lambda i: (i, 0)),
            pl.BlockSpec((1, H), lambda i: (0, 0)),
            pl.BlockSpec((1, H), lambda i: (0, 0)),
            pl.BlockSpec((1, H), lambda i: (0, 0)),
            pl.BlockSpec((1, H), lambda i: (0, 0)),
            pl.BlockSpec((_B, H), lambda i: (i, 0)),
        ],
        out_specs=pl.BlockSpec((_B, H), lambda i: (i, 0)),
        out_shape=jax.ShapeDtypeStruct((N, H), _f32),
    )(y, s, q, g, be, hp)


def _k5_body(xa, xb, bt, linW, linb, out_o, Pa, Pb, C8):
    i = pl.program_id(0)

    @pl.when(i == 0)
    def _():
        Pa[...] = jnp.zeros_like(Pa)
        Pb[...] = jnp.zeros_like(Pb)
        C8[...] = jnp.zeros_like(C8)

    oh = (bt[...] == lax.broadcasted_iota(jnp.int32, (1, G), 1)).astype(_f32)
    dims = (((0,), (0,)), ((), ()))
    Pa[...] += lax.dot_general(oh, xa[...], dims, preferred_element_type=_f32)
    Pb[...] += lax.dot_general(oh, xb[...], dims, preferred_element_type=_f32)
    C8[...] += lax.dot_general(oh, jnp.ones((_B, 8), _f32), dims,
                               preferred_element_type=_f32)

    @pl.when(i == 9)
    def _():
        cnt = jnp.maximum(C8[...][:, 0:1], 1.0)
        z = jnp.concatenate([Pa[...] / cnt, Pb[...] / cnt], axis=1)
        out_o[...] = jnp.dot(z, linW[...], preferred_element_type=_f32) + linb[...]


def _tc_pool(xa, xb, bt, linW, linb):
    return pl.pallas_call(
        _k5_body,
        grid=(10,),
        in_specs=[
            pl.BlockSpec((_B, H), lambda i: (i, 0)),
            pl.BlockSpec((_B, H), lambda i: (i, 0)),
            pl.BlockSpec((_B, 1), lambda i: (i, 0)),
            pl.BlockSpec((2 * H, OUT), lambda i: (0, 0)),
            pl.BlockSpec((1, OUT), lambda i: (0, 0)),
        ],
        out_specs=pl.BlockSpec((G, OUT), lambda i: (0, 0)),
        out_shape=jax.ShapeDtypeStruct((G, OUT), _f32),
        scratch_shapes=[
            pltpu.VMEM((G, H), _f32),
            pltpu.VMEM((G, H), _f32),
            pltpu.VMEM((G, 8), _f32),
        ],
    )(xa, xb, bt, linW, linb)


def _pad1(a, length, value):
    return jnp.concatenate(
        [a, jnp.full((length - a.shape[0],), value, a.dtype)])


def kernel(x, edge_index, cycle_index, batch, atom_tables, W0, b0, g0, be0,
           W1, b1, g1, be1, W2, b2, g2, be2, Wc, bc, gc, bec, lin_W, lin_b):
    i32 = jnp.int32
    idx_emb = (x.astype(i32).T + 119 * jnp.arange(9, dtype=i32)[:, None]).reshape(-1)
    node_rep = jnp.tile(jnp.arange(N, dtype=i32), 9)
    ep = _pad1(idx_emb, EPM, 0)
    emb_src2 = jnp.stack([ep, ep + TROWS]).reshape(32 * SUPM, NB, CH)
    emb_dst = _pad1(node_rep, EPM, N).reshape(16 * SUPM, NB, CH)

    def edge_prep(ei):
        sp = _pad1(ei[0].astype(i32), EP, 0)
        src2 = jnp.stack([sp, sp + N]).reshape(32 * SUPE, NB, CH)
        dp = _pad1(ei[1].astype(i32), EP, N)
        dstp = dp.reshape(16 * SUPE, NB, CH)
        dstd = dp.reshape(16 * SUPD, 8, DCH)
        return src2, dstp, dstd

    src2_e, dstp_e, dstd_e = edge_prep(edge_index)
    src2_c, dstp_c, dstd_c = edge_prep(cycle_index)
    tcol = atom_tables.reshape(TROWS, 2, 128).transpose(1, 0, 2).reshape(2 * TROWS, 128)

    x0c = _sc_emb(tcol, emb_src2, emb_dst)
    degc = _sc_deg(dstd_e, dstd_c)
    x0l, x0r = x0c[0:N], x0c[NROWS:NROWS + N]
    hist_e = degc[0:N, 0:1]
    hist_c = degc[NROWS:NROWS + N, 0:1]

    h0, dinv_e, dinv_c, hws0, hwsc = _tc_prep(x0l, x0r, hist_e, hist_c, W0, Wc)

    def agg(hws, src2, dstp):
        a = _sc_agg(hws, src2, dstp)
        return a[0:N], a[NROWS:NROWS + N]

    b0r, g0r, be0r = b0.reshape(1, H), g0.reshape(1, H), be0.reshape(1, H)
    b1r, g1r, be1r = b1.reshape(1, H), g1.reshape(1, H), be1.reshape(1, H)
    b2r, g2r, be2r = b2.reshape(1, H), g2.reshape(1, H), be2.reshape(1, H)
    bcr, gcr, becr = bc.reshape(1, H), gc.reshape(1, H), bec.reshape(1, H)

    a0l, a0r = agg(hws0, src2_e, dstp_e)
    y0, s0, q0 = _tc_stats(a0l, a0r, hws0[0:N], hws0[N:2 * N], dinv_e, b0r)
    h1, hws1 = _tc_norm_next(y0, s0, q0, g0r, be0r, h0, W1, dinv_e)
    a1l, a1r = agg(hws1, src2_e, dstp_e)
    y1, s1, q1 = _tc_stats(a1l, a1r, hws1[0:N], hws1[N:2 * N], dinv_e, b1r)
    h2, hws2 = _tc_norm_next(y1, s1, q1, g1r, be1r, h1, W2, dinv_e)
    a2l, a2r = agg(hws2, src2_e, dstp_e)
    y2, s2, q2 = _tc_stats(a2l, a2r, hws2[0:N], hws2[N:2 * N], dinv_e, b2r)
    x_out_a = _tc_norm_final(y2, s2, q2, g2r, be2r, h2)
    acl, acr = agg(hwsc, src2_c, dstp_c)
    yc, scs, qc = _tc_stats(acl, acr, hwsc[0:N], hwsc[N:2 * N], dinv_c, bcr)
    x_out_b = _tc_norm_final(yc, scs, qc, gcr, becr, h0)

    bt = batch.astype(i32).reshape(N, 1)
    return _tc_pool(x_out_a, x_out_b, bt, lin_W, lin_b.reshape(1, OUT))


# speedup vs baseline: 1.3010x; 1.0099x over previous
import functools

import jax
import jax.numpy as jnp
from jax import lax
from jax.experimental import pallas as pl
from jax.experimental.pallas import tpu as pltpu
from jax.experimental.pallas import tpu_sc as plsc

N = 10000
H = 256
G = 256
OUT = 128
NROWS = 10240
CH = 128
NB = 2
SUPE = 40
EP = 16 * SUPE * NB * CH
SUPM = 24
EPM = 16 * SUPM * NB * CH
SUPD = 10
DCH = 128
TROWS = 9 * 119

_f32 = jnp.float32
_mesh = plsc.VectorSubcoreMesh(core_axis_name="c", subcore_axis_name="s")
_sc_params = pltpu.CompilerParams(use_tc_tiling_on_sc=False)


def _fill2d(ref, nrows, value):
    def body(r, carry):
        ref[r, pl.ds(0, 16)] = jnp.full((16,), value, _f32)
        return carry
    lax.fori_loop(0, nrows, body, 0)


def _zero_buf(ref):
    def body(r, carry):
        for c in range(8):
            ref[r, pl.ds(c * 16, 16)] = jnp.zeros((16,), _f32)
        return carry
    lax.fori_loop(0, CH, body, 0)


def _gather_scatter_loop(table, srch, dsth, idxb, dstb, bufs, acc,
                         semg, sems, cid, sid, nsup):
    wrow = (cid * 16 + sid) * nsup
    drow = sid * nsup

    def super_body(s, carry):
        pltpu.sync_copy(srch.at[wrow + s], idxb)
        gs = [pltpu.async_copy(table.at[idxb.at[b]], bufs.at[b], semg)
              for b in range(NB)]
        pltpu.sync_copy(dsth.at[drow + s], dstb)
        ss = []
        for b in range(NB):
            gs[b].wait()
            ss.append(pltpu.async_copy(bufs.at[b], acc.at[dstb.at[b]], sems,
                                       add=True))
        for h in ss:
            h.wait()
        return carry
    lax.fori_loop(0, nsup, super_body, 0)


def _zero_acc(bufs, acc, sid):
    _zero_buf(bufs.at[0])
    for j in range(5):
        pltpu.sync_copy(bufs.at[0], acc.at[pl.ds(sid * 640 + j * CH, CH)])


def _writeout(acc, out, cid, sid):
    for j in range(5):
        pltpu.sync_copy(acc.at[pl.ds(sid * 640 + j * CH, CH)],
                        out.at[pl.ds(cid * NROWS + sid * 640 + j * CH, CH)])


@functools.partial(
    pl.kernel,
    out_type=jax.ShapeDtypeStruct((2 * NROWS, 128), _f32),
    mesh=_mesh,
    compiler_params=_sc_params,
    scratch_types=[
        pltpu.VMEM((NB, CH), jnp.int32),
        pltpu.VMEM((NB, CH), jnp.int32),
        pltpu.VMEM((NB, CH, 128), _f32),
        pltpu.VMEM_SHARED((NROWS, 128), _f32),
        pltpu.SemaphoreType.DMA,
        pltpu.SemaphoreType.DMA,
    ],
)
def _sc_emb(tcol, esrc2, edst, x0_out, idxb, dstb, bufs, acc, semg, sems):
    cid = lax.axis_index("c")
    sid = lax.axis_index("s")
    _zero_acc(bufs, acc, sid)
    plsc.subcore_barrier()
    _gather_scatter_loop(tcol, esrc2, edst, idxb, dstb, bufs, acc,
                         semg, sems, cid, sid, SUPM)
    plsc.subcore_barrier()
    _writeout(acc, x0_out, cid, sid)


@functools.partial(
    pl.kernel,
    out_type=jax.ShapeDtypeStruct((2 * NROWS, 16), _f32),
    mesh=_mesh,
    compiler_params=_sc_params,
    scratch_types=[
        pltpu.VMEM((8, DCH), jnp.int32),
        pltpu.VMEM((DCH, 16), _f32),
        pltpu.VMEM((640, 16), _f32),
        pltpu.VMEM_SHARED((NROWS, 16), _f32),
        pltpu.SemaphoreType.DMA,
    ],
)
def _sc_deg(dste, dstc, deg_out, dstb8, ones16, z16, accdeg, semd):
    cid = lax.axis_index("c")
    sid = lax.axis_index("s")
    _fill2d(z16, 640, 0.0)
    _fill2d(ones16, DCH, 1.0)
    pltpu.sync_copy(z16, accdeg.at[pl.ds(sid * 640, 640)])
    plsc.subcore_barrier()

    def deg_super(s, carry):
        @pl.when(cid == 0)
        def _():
            pltpu.sync_copy(dste.at[sid * SUPD + s], dstb8)

        @pl.when(cid == 1)
        def _():
            pltpu.sync_copy(dstc.at[sid * SUPD + s], dstb8)

        hs = [pltpu.async_copy(ones16, accdeg.at[dstb8.at[b]], semd, add=True)
              for b in range(8)]
        for h in hs:
            h.wait()
        return carry
    lax.fori_loop(0, SUPD, deg_super, 0)

    plsc.subcore_barrier()
    pltpu.sync_copy(accdeg.at[pl.ds(sid * 640, 640)],
                    deg_out.at[pl.ds(cid * NROWS + sid * 640, 640)])


@functools.partial(
    pl.kernel,
    out_type=jax.ShapeDtypeStruct((2 * NROWS, 128), _f32),
    mesh=_mesh,
    compiler_params=_sc_params,
    scratch_types=[
        pltpu.VMEM((NB, CH), jnp.int32),
        pltpu.VMEM((NB, CH), jnp.int32),
        pltpu.VMEM((NB, CH, 128), _f32),
        pltpu.VMEM_SHARED((NROWS, 128), _f32),
        pltpu.SemaphoreType.DMA,
        pltpu.SemaphoreType.DMA,
    ],
)
def _sc_agg(table, src2, dstp, out, idxb, dstb, bufs, acc, semg, sems):
    cid = lax.axis_index("c")
    sid = lax.axis_index("s")
    _zero_acc(bufs, acc, sid)
    plsc.subcore_barrier()
    _gather_scatter_loop(table, src2, dstp, idxb, dstb, bufs, acc,
                         semg, sems, cid, sid, SUPE)
    plsc.subcore_barrier()
    _writeout(acc, out, cid, sid)


_B = 1000


def _k1_body(x0l, x0r, he, hc, W0, Wc, h0_o, de_o, dc_o, hws0_o, hwsc_o):
    h = jnp.concatenate([x0l[...], x0r[...]], axis=1)
    h0_o[...] = h
    de = lax.rsqrt(he[...] + 1.0)
    dc = lax.rsqrt(hc[...] + 1.0)
    de_o[...] = de
    dc_o[...] = dc
    hws0_o[...] = (jnp.dot(h, W0[...], preferred_element_type=_f32) * de
                   ).reshape(_B, 2, 128)
    hwsc_o[...] = (jnp.dot(h, Wc[...], preferred_element_type=_f32) * dc
                   ).reshape(_B, 2, 128)


def _tc_prep(x0l, x0r, hist_e, hist_c, W0, Wc):
    return pl.pallas_call(
        _k1_body,
        grid=(10,),
        in_specs=[
            pl.BlockSpec((_B, 128), lambda i: (i, 0)),
            pl.BlockSpec((_B, 128), lambda i: (i, 0)),
            pl.BlockSpec((_B, 1), lambda i: (i, 0)),
            pl.BlockSpec((_B, 1), lambda i: (i, 0)),
            pl.BlockSpec((H, H), lambda i: (0, 0)),
            pl.BlockSpec((H, H), lambda i: (0, 0)),
        ],
        out_specs=[
            pl.BlockSpec((_B, H), lambda i: (i, 0)),
            pl.BlockSpec((_B, 1), lambda i: (i, 0)),
            pl.BlockSpec((_B, 1), lambda i: (i, 0)),
            pl.BlockSpec((_B, 2, 128), lambda i: (i, 0, 0)),
            pl.BlockSpec((_B, 2, 128), lambda i: (i, 0, 0)),
        ],
        out_shape=[
            jax.ShapeDtypeStruct((N, H), _f32),
            jax.ShapeDtypeStruct((N, 1), _f32),
            jax.ShapeDtypeStruct((N, 1), _f32),
            jax.ShapeDtypeStruct((N, 2, 128), _f32),
            jax.ShapeDtypeStruct((N, 2, 128), _f32),
        ],
    )(x0l, x0r, hist_e, hist_c, W0, Wc)


def _k2_body(al, ar, hws3, dv, b, y_o, s_o, q_o):
    i = pl.program_id(0)
    agg = jnp.concatenate([al[...], ar[...]], axis=1)
    hws = hws3[...].reshape(_B, H)
    y = dv[...] * (agg + hws) + b[...]
    y_o[...] = y

    @pl.when(i == 0)
    def _():
        s_o[...] = jnp.zeros_like(s_o)
        q_o[...] = jnp.zeros_like(q_o)

    s_o[...] += jnp.sum(y, axis=0, keepdims=True)
    q_o[...] += jnp.sum(y * y, axis=0, keepdims=True)


def _tc_stats(al, ar, hws3, dv, b):
    return pl.pallas_call(
        _k2_body,
        grid=(10,),
        in_specs=[
            pl.BlockSpec((_B, 128), lambda i: (i, 0)),
            pl.BlockSpec((_B, 128), lambda i: (i, 0)),
            pl.BlockSpec((_B, 2, 128), lambda i: (i, 0, 0)),
            pl.BlockSpec((_B, 1), lambda i: (i, 0)),
            pl.BlockSpec((1, H), lambda i: (0, 0)),
        ],
        out_specs=[
            pl.BlockSpec((_B, H), lambda i: (i, 0)),
            pl.BlockSpec((1, H), lambda i: (0, 0)),
            pl.BlockSpec((1, H), lambda i: (0, 0)),
        ],
        out_shape=[
            jax.ShapeDtypeStruct((N, H), _f32),
            jax.ShapeDtypeStruct((1, H), _f32),
            jax.ShapeDtypeStruct((1, H), _f32),
        ],
    )(al, ar, hws3, dv, b)


def _k3_body(y, s, q, g, be, hp, Wn, dv, hn_o, hws_o):
    mu = s[...] / N
    var = q[...] / N - mu * mu
    sc = g[...] * lax.rsqrt(var + 1e-5)
    sh = be[...] - mu * sc
    hn = jnp.maximum(y[...] * sc + sh, 0.0) + hp[...]
    hn_o[...] = hn
    hws_o[...] = (jnp.dot(hn, Wn[...], preferred_element_type=_f32) * dv[...]
                  ).reshape(_B, 2, 128)


def _tc_norm_next(y, s, q, g, be, hp, Wn, dv):
    return pl.pallas_call(
        _k3_body,
        grid=(10,),
        in_specs=[
            pl.BlockSpec((_B, H), lambda i: (i, 0)),
            pl.BlockSpec((1, H), lambda i: (0, 0)),
            pl.BlockSpec((1, H), lambda i: (0, 0)),
            pl.BlockSpec((1, H), lambda i: (0, 0)),
            pl.BlockSpec((1, H), lambda i: (0, 0)),
            pl.BlockSpec((_B, H), lambda i: (i, 0)),
            pl.BlockSpec((H, H), lambda i: (0, 0)),
            pl.BlockSpec((_B, 1), lambda i: (i, 0)),
        ],
        out_specs=[
            pl.BlockSpec((_B, H), lambda i: (i, 0)),
            pl.BlockSpec((_B, 2, 128), lambda i: (i, 0, 0)),
        ],
        out_shape=[
            jax.ShapeDtypeStruct((N, H), _f32),
            jax.ShapeDtypeStruct((N, 2, 128), _f32),
        ],
    )(y, s, q, g, be, hp, Wn, dv)


def _k4_body(y, s, q, g, be, hp, x_o):
    mu = s[...] / N
    var = q[...] / N - mu * mu
    sc = g[...] * lax.rsqrt(var + 1e-5)
    sh = be[...] - mu * sc
    x_o[...] = jnp.maximum(y[...] * sc + sh, 0.0) + hp[...]


def _tc_norm_final(y, s, q, g, be, hp):
    return pl.pallas_call(
        _k4_body,
        grid=(10,),
        in_specs=[
            pl.BlockSpec((_B, H), lambda i: (i, 0)),
            pl.BlockSpec((1, H), lambda i: (0, 0)),
            pl.BlockSpec((1, H), lambda i: (0, 0)),
            pl.BlockSpec((1, H), lambda i: (0, 0)),
            pl.BlockSpec((1, H), lambda i: (0, 0)),
            pl.BlockSpec((_B, H), lambda i: (i, 0)),
        ],
        out_specs=pl.BlockSpec((_B, H), lambda i: (i, 0)),
        out_shape=jax.ShapeDtypeStruct((N, H), _f32),
    )(y, s, q, g, be, hp)


def _k5_body(xa, xb, bt, linW, linb, out_o, Pa, Pb, C8):
    i = pl.program_id(0)

    @pl.when(i == 0)
    def _():
        Pa[...] = jnp.zeros_like(Pa)
        Pb[...] = jnp.zeros_like(Pb)
        C8[...] = jnp.zeros_like(C8)

    oh = (bt[...] == lax.broadcasted_iota(jnp.int32, (1, G), 1)).astype(_f32)
    dims = (((0,), (0,)), ((), ()))
    Pa[...] += lax.dot_general(oh, xa[...], dims, preferred_element_type=_f32)
    Pb[...] += lax.dot_general(oh, xb[...], dims, preferred_element_type=_f32)
    C8[...] += lax.dot_general(oh, jnp.ones((_B, 8), _f32), dims,
                               preferred_element_type=_f32)

    @pl.when(i == 9)
    def _():
        cnt = jnp.maximum(C8[...][:, 0:1], 1.0)
        z = jnp.concatenate([Pa[...] / cnt, Pb[...] / cnt], axis=1)
        out_o[...] = jnp.dot(z, linW[...], preferred_element_type=_f32) + linb[...]


def _tc_pool(xa, xb, bt, linW, linb):
    return pl.pallas_call(
        _k5_body,
        grid=(10,),
        in_specs=[
            pl.BlockSpec((_B, H), lambda i: (i, 0)),
            pl.BlockSpec((_B, H), lambda i: (i, 0)),
            pl.BlockSpec((_B, 1), lambda i: (i, 0)),
            pl.BlockSpec((2 * H, OUT), lambda i: (0, 0)),
            pl.BlockSpec((1, OUT), lambda i: (0, 0)),
        ],
        out_specs=pl.BlockSpec((G, OUT), lambda i: (0, 0)),
        out_shape=jax.ShapeDtypeStruct((G, OUT), _f32),
        scratch_shapes=[
            pltpu.VMEM((G, H), _f32),
            pltpu.VMEM((G, H), _f32),
            pltpu.VMEM((G, 8), _f32),
        ],
    )(xa, xb, bt, linW, linb)


def _pad1(a, length, value):
    return jnp.concatenate(
        [a, jnp.full((length - a.shape[0],), value, a.dtype)])


def kernel(x, edge_index, cycle_index, batch, atom_tables, W0, b0, g0, be0,
           W1, b1, g1, be1, W2, b2, g2, be2, Wc, bc, gc, bec, lin_W, lin_b):
    i32 = jnp.int32
    idx_emb = (x.astype(i32).T + 119 * jnp.arange(9, dtype=i32)[:, None]).reshape(-1)
    node_rep = jnp.tile(jnp.arange(N, dtype=i32), 9)
    ep = _pad1(idx_emb, EPM, 0)
    emb_src2 = jnp.stack([2 * ep, 2 * ep + 1]).reshape(32 * SUPM, NB, CH)
    emb_dst = _pad1(node_rep, EPM, N).reshape(16 * SUPM, NB, CH)

    def edge_prep(ei):
        sp = _pad1(ei[0].astype(i32), EP, 0)
        src2 = jnp.stack([2 * sp, 2 * sp + 1]).reshape(32 * SUPE, NB, CH)
        dp = _pad1(ei[1].astype(i32), EP, N)
        dstp = dp.reshape(16 * SUPE, NB, CH)
        dstd = dp.reshape(16 * SUPD, 8, DCH)
        return src2, dstp, dstd

    src2_e, dstp_e, dstd_e = edge_prep(edge_index)
    src2_c, dstp_c, dstd_c = edge_prep(cycle_index)
    tcol = atom_tables.reshape(2 * TROWS, 128)

    x0c = _sc_emb(tcol, emb_src2, emb_dst)
    degc = _sc_deg(dstd_e, dstd_c)
    x0l, x0r = x0c[0:N], x0c[NROWS:NROWS + N]
    hist_e = degc[0:N, 0:1]
    hist_c = degc[NROWS:NROWS + N, 0:1]

    h0, dinv_e, dinv_c, hws0, hwsc = _tc_prep(x0l, x0r, hist_e, hist_c, W0, Wc)

    def agg(hws3, src2, dstp):
        a = _sc_agg(hws3.reshape(2 * N, 128), src2, dstp)
        return a[0:N], a[NROWS:NROWS + N]

    b0r, g0r, be0r = b0.reshape(1, H), g0.reshape(1, H), be0.reshape(1, H)
    b1r, g1r, be1r = b1.reshape(1, H), g1.reshape(1, H), be1.reshape(1, H)
    b2r, g2r, be2r = b2.reshape(1, H), g2.reshape(1, H), be2.reshape(1, H)
    bcr, gcr, becr = bc.reshape(1, H), gc.reshape(1, H), bec.reshape(1, H)

    a0l, a0r = agg(hws0, src2_e, dstp_e)
    y0, s0, q0 = _tc_stats(a0l, a0r, hws0, dinv_e, b0r)
    h1, hws1 = _tc_norm_next(y0, s0, q0, g0r, be0r, h0, W1, dinv_e)
    a1l, a1r = agg(hws1, src2_e, dstp_e)
    y1, s1, q1 = _tc_stats(a1l, a1r, hws1, dinv_e, b1r)
    h2, hws2 = _tc_norm_next(y1, s1, q1, g1r, be1r, h1, W2, dinv_e)
    a2l, a2r = agg(hws2, src2_e, dstp_e)
    y2, s2, q2 = _tc_stats(a2l, a2r, hws2, dinv_e, b2r)
    x_out_a = _tc_norm_final(y2, s2, q2, g2r, be2r, h2)
    acl, acr = agg(hwsc, src2_c, dstp_c)
    yc, scs, qc = _tc_stats(acl, acr, hwsc, dinv_c, bcr)
    x_out_b = _tc_norm_final(yc, scs, qc, gcr, becr, h0)

    bt = batch.astype(i32).reshape(N, 1)
    return _tc_pool(x_out_a, x_out_b, bt, lin_W, lin_b.reshape(1, OUT))


# speedup vs baseline: 1.6398x; 1.2604x over previous
import functools

import jax
import jax.numpy as jnp
from jax import lax
from jax.experimental import pallas as pl
from jax.experimental.pallas import tpu as pltpu
from jax.experimental.pallas import tpu_sc as plsc

N = 10000
H = 256
G = 256
OUT = 128
NROWS = 10240
CH = 128
NB = 2
SUPE = 40
EP = 16 * SUPE * NB * CH
CHM = 64
SUPM = 48
EPM = 16 * SUPM * NB * CHM
TPAD = 2144
SUPD = 10
DCH = 128
TROWS = 9 * 119

_f32 = jnp.float32
_mesh = plsc.VectorSubcoreMesh(core_axis_name="c", subcore_axis_name="s")
_sc_params = pltpu.CompilerParams(use_tc_tiling_on_sc=False)


def _fill2d(ref, nrows, value):
    def body(r, carry):
        ref[r, pl.ds(0, 16)] = jnp.full((16,), value, _f32)
        return carry
    lax.fori_loop(0, nrows, body, 0)


def _zero_buf(ref):
    def body(r, carry):
        for c in range(8):
            ref[r, pl.ds(c * 16, 16)] = jnp.zeros((16,), _f32)
        return carry
    lax.fori_loop(0, ref.shape[0], body, 0)


def _gather_scatter_loop(table, srch, dsth, idxb, dstb, bufs, acc,
                         semg, sems, cid, sid, nsup):
    wrow = (cid * 16 + sid) * nsup
    drow = sid * nsup

    def super_body(s, carry):
        pltpu.sync_copy(srch.at[wrow + s], idxb)
        gs = [pltpu.async_copy(table.at[idxb.at[b]], bufs.at[b], semg)
              for b in range(NB)]
        pltpu.sync_copy(dsth.at[drow + s], dstb)
        ss = []
        for b in range(NB):
            gs[b].wait()
            ss.append(pltpu.async_copy(bufs.at[b], acc.at[dstb.at[b]], sems,
                                       add=True))
        for h in ss:
            h.wait()
        return carry
    lax.fori_loop(0, nsup, super_body, 0)


def _zero_acc(bufs, acc, sid):
    _zero_buf(bufs.at[0])
    for j in range(5):
        pltpu.sync_copy(bufs.at[0], acc.at[pl.ds(sid * 640 + j * CH, CH)])


def _writeout(acc, out, cid, sid):
    for j in range(5):
        pltpu.sync_copy(acc.at[pl.ds(sid * 640 + j * CH, CH)],
                        out.at[pl.ds(cid * NROWS + sid * 640 + j * CH, CH)])


@functools.partial(
    pl.kernel,
    out_type=jax.ShapeDtypeStruct((2 * NROWS, 128), _f32),
    mesh=_mesh,
    compiler_params=_sc_params,
    scratch_types=[
        pltpu.VMEM((NB, CHM), jnp.int32),
        pltpu.VMEM((NB, CHM), jnp.int32),
        pltpu.VMEM((NB, CHM, 128), _f32),
        pltpu.VMEM_SHARED((NROWS, 128), _f32),
        pltpu.VMEM_SHARED((TPAD, 128), _f32),
        pltpu.SemaphoreType.DMA,
        pltpu.SemaphoreType.DMA,
    ],
)
def _sc_emb(tcol, esrc2, edst, x0_out, idxb, dstb, bufs, acc, tsh,
            semg, sems):
    cid = lax.axis_index("c")
    sid = lax.axis_index("s")
    rows = TPAD // 16
    pltpu.sync_copy(tcol.at[pl.ds(sid * rows, rows)],
                    tsh.at[pl.ds(sid * rows, rows)])
    _zero_buf(bufs.at[0])
    for j in range(10):
        pltpu.sync_copy(bufs.at[0], acc.at[pl.ds(sid * 640 + j * CHM, CHM)])
    plsc.subcore_barrier()
    _gather_scatter_loop(tsh, esrc2, edst, idxb, dstb, bufs, acc,
                         semg, sems, cid, sid, SUPM)
    plsc.subcore_barrier()
    _writeout(acc, x0_out, cid, sid)


@functools.partial(
    pl.kernel,
    out_type=jax.ShapeDtypeStruct((2 * NROWS, 16), _f32),
    mesh=_mesh,
    compiler_params=_sc_params,
    scratch_types=[
        pltpu.VMEM((8, DCH), jnp.int32),
        pltpu.VMEM((DCH, 16), _f32),
        pltpu.VMEM((640, 16), _f32),
        pltpu.VMEM_SHARED((NROWS, 16), _f32),
        pltpu.SemaphoreType.DMA,
    ],
)
def _sc_deg(dste, dstc, deg_out, dstb8, ones16, z16, accdeg, semd):
    cid = lax.axis_index("c")
    sid = lax.axis_index("s")
    _fill2d(z16, 640, 0.0)
    _fill2d(ones16, DCH, 1.0)
    pltpu.sync_copy(z16, accdeg.at[pl.ds(sid * 640, 640)])
    plsc.subcore_barrier()

    def deg_super(s, carry):
        @pl.when(cid == 0)
        def _():
            pltpu.sync_copy(dste.at[sid * SUPD + s], dstb8)

        @pl.when(cid == 1)
        def _():
            pltpu.sync_copy(dstc.at[sid * SUPD + s], dstb8)

        hs = [pltpu.async_copy(ones16, accdeg.at[dstb8.at[b]], semd, add=True)
              for b in range(8)]
        for h in hs:
            h.wait()
        return carry
    lax.fori_loop(0, SUPD, deg_super, 0)

    plsc.subcore_barrier()
    pltpu.sync_copy(accdeg.at[pl.ds(sid * 640, 640)],
                    deg_out.at[pl.ds(cid * NROWS + sid * 640, 640)])


@functools.partial(
    pl.kernel,
    out_type=jax.ShapeDtypeStruct((2 * NROWS, 128), _f32),
    mesh=_mesh,
    compiler_params=_sc_params,
    scratch_types=[
        pltpu.VMEM((NB, CH), jnp.int32),
        pltpu.VMEM((NB, CH), jnp.int32),
        pltpu.VMEM((NB, CH, 128), _f32),
        pltpu.VMEM_SHARED((NROWS, 128), _f32),
        pltpu.SemaphoreType.DMA,
        pltpu.SemaphoreType.DMA,
    ],
)
def _sc_agg(table, src2, dstp, out, idxb, dstb, bufs, acc, semg, sems):
    cid = lax.axis_index("c")
    sid = lax.axis_index("s")
    _zero_acc(bufs, acc, sid)
    plsc.subcore_barrier()
    _gather_scatter_loop(table, src2, dstp, idxb, dstb, bufs, acc,
                         semg, sems, cid, sid, SUPE)
    plsc.subcore_barrier()
    _writeout(acc, out, cid, sid)


_B = 1000


def _k1_body(x0l, x0r, he, hc, W0, Wc, h0_o, de_o, dc_o, hws0_o, hwsc_o):
    h = jnp.concatenate([x0l[...], x0r[...]], axis=1)
    h0_o[...] = h
    de = lax.rsqrt(he[...] + 1.0)
    dc = lax.rsqrt(hc[...] + 1.0)
    de_o[...] = de
    dc_o[...] = dc
    hws0_o[...] = (jnp.dot(h, W0[...], preferred_element_type=_f32) * de
                   ).reshape(_B, 2, 128)
    hwsc_o[...] = (jnp.dot(h, Wc[...], preferred_element_type=_f32) * dc
                   ).reshape(_B, 2, 128)


def _tc_prep(x0l, x0r, hist_e, hist_c, W0, Wc):
    return pl.pallas_call(
        _k1_body,
        grid=(10,),
        in_specs=[
            pl.BlockSpec((_B, 128), lambda i: (i, 0)),
            pl.BlockSpec((_B, 128), lambda i: (i, 0)),
            pl.BlockSpec((_B, 1), lambda i: (i, 0)),
            pl.BlockSpec((_B, 1), lambda i: (i, 0)),
            pl.BlockSpec((H, H), lambda i: (0, 0)),
            pl.BlockSpec((H, H), lambda i: (0, 0)),
        ],
        out_specs=[
            pl.BlockSpec((_B, H), lambda i: (i, 0)),
            pl.BlockSpec((_B, 1), lambda i: (i, 0)),
            pl.BlockSpec((_B, 1), lambda i: (i, 0)),
            pl.BlockSpec((_B, 2, 128), lambda i: (i, 0, 0)),
            pl.BlockSpec((_B, 2, 128), lambda i: (i, 0, 0)),
        ],
        out_shape=[
            jax.ShapeDtypeStruct((N, H), _f32),
            jax.ShapeDtypeStruct((N, 1), _f32),
            jax.ShapeDtypeStruct((N, 1), _f32),
            jax.ShapeDtypeStruct((N, 2, 128), _f32),
            jax.ShapeDtypeStruct((N, 2, 128), _f32),
        ],
    )(x0l, x0r, hist_e, hist_c, W0, Wc)


def _k2_body(al, ar, hws3, dv, b, y_o, s_o, q_o):
    i = pl.program_id(0)
    agg = jnp.concatenate([al[...], ar[...]], axis=1)
    hws = hws3[...].reshape(_B, H)
    y = dv[...] * (agg + hws) + b[...]
    y_o[...] = y

    @pl.when(i == 0)
    def _():
        s_o[...] = jnp.zeros_like(s_o)
        q_o[...] = jnp.zeros_like(q_o)

    s_o[...] += jnp.sum(y, axis=0, keepdims=True)
    q_o[...] += jnp.sum(y * y, axis=0, keepdims=True)


def _tc_stats(al, ar, hws3, dv, b):
    return pl.pallas_call(
        _k2_body,
        grid=(10,),
        in_specs=[
            pl.BlockSpec((_B, 128), lambda i: (i, 0)),
            pl.BlockSpec((_B, 128), lambda i: (i, 0)),
            pl.BlockSpec((_B, 2, 128), lambda i: (i, 0, 0)),
            pl.BlockSpec((_B, 1), lambda i: (i, 0)),
            pl.BlockSpec((1, H), lambda i: (0, 0)),
        ],
        out_specs=[
            pl.BlockSpec((_B, H), lambda i: (i, 0)),
            pl.BlockSpec((1, H), lambda i: (0, 0)),
            pl.BlockSpec((1, H), lambda i: (0, 0)),
        ],
        out_shape=[
            jax.ShapeDtypeStruct((N, H), _f32),
            jax.ShapeDtypeStruct((1, H), _f32),
            jax.ShapeDtypeStruct((1, H), _f32),
        ],
    )(al, ar, hws3, dv, b)


def _k3_body(y, s, q, g, be, hp, Wn, dv, hn_o, hws_o):
    mu = s[...] / N
    var = q[...] / N - mu * mu
    sc = g[...] * lax.rsqrt(var + 1e-5)
    sh = be[...] - mu * sc
    hn = jnp.maximum(y[...] * sc + sh, 0.0) + hp[...]
    hn_o[...] = hn
    hws_o[...] = (jnp.dot(hn, Wn[...], preferred_element_type=_f32) * dv[...]
                  ).reshape(_B, 2, 128)


def _tc_norm_next(y, s, q, g, be, hp, Wn, dv):
    return pl.pallas_call(
        _k3_body,
        grid=(10,),
        in_specs=[
            pl.BlockSpec((_B, H), lambda i: (i, 0)),
            pl.BlockSpec((1, H), lambda i: (0, 0)),
            pl.BlockSpec((1, H), lambda i: (0, 0)),
            pl.BlockSpec((1, H), lambda i: (0, 0)),
            pl.BlockSpec((1, H), lambda i: (0, 0)),
            pl.BlockSpec((_B, H), lambda i: (i, 0)),
            pl.BlockSpec((H, H), lambda i: (0, 0)),
            pl.BlockSpec((_B, 1), lambda i: (i, 0)),
        ],
        out_specs=[
            pl.BlockSpec((_B, H), lambda i: (i, 0)),
            pl.BlockSpec((_B, 2, 128), lambda i: (i, 0, 0)),
        ],
        out_shape=[
            jax.ShapeDtypeStruct((N, H), _f32),
            jax.ShapeDtypeStruct((N, 2, 128), _f32),
        ],
    )(y, s, q, g, be, hp, Wn, dv)


def _k4_body(y, s, q, g, be, hp, x_o):
    mu = s[...] / N
    var = q[...] / N - mu * mu
    sc = g[...] * lax.rsqrt(var + 1e-5)
    sh = be[...] - mu * sc
    x_o[...] = jnp.maximum(y[...] * sc + sh, 0.0) + hp[...]


def _tc_norm_final(y, s, q, g, be, hp):
    return pl.pallas_call(
        _k4_body,
        grid=(10,),
        in_specs=[
            pl.BlockSpec((_B, H), lambda i: (i, 0)),
            pl.BlockSpec((1, H), lambda i: (0, 0)),
            pl.BlockSpec((1, H), lambda i: (0, 0)),
            pl.BlockSpec((1, H), lambda i: (0, 0)),
            pl.BlockSpec((1, H), lambda i: (0, 0)),
            pl.BlockSpec((_B, H), lambda i: (i, 0)),
        ],
        out_specs=pl.BlockSpec((_B, H), lambda i: (i, 0)),
        out_shape=jax.ShapeDtypeStruct((N, H), _f32),
    )(y, s, q, g, be, hp)


def _k5_body(xa, xb, bt, linW, linb, out_o, Pa, Pb, C8):
    i = pl.program_id(0)

    @pl.when(i == 0)
    def _():
        Pa[...] = jnp.zeros_like(Pa)
        Pb[...] = jnp.zeros_like(Pb)
        C8[...] = jnp.zeros_like(C8)

    oh = (bt[...] == lax.broadcasted_iota(jnp.int32, (1, G), 1)).astype(_f32)
    dims = (((0,), (0,)), ((), ()))
    Pa[...] += lax.dot_general(oh, xa[...], dims, preferred_element_type=_f32)
    Pb[...] += lax.dot_general(oh, xb[...], dims, preferred_element_type=_f32)
    C8[...] += lax.dot_general(oh, jnp.ones((_B, 8), _f32), dims,
                               preferred_element_type=_f32)

    @pl.when(i == 9)
    def _():
        cnt = jnp.maximum(C8[...][:, 0:1], 1.0)
        z = jnp.concatenate([Pa[...] / cnt, Pb[...] / cnt], axis=1)
        out_o[...] = jnp.dot(z, linW[...], preferred_element_type=_f32) + linb[...]


def _tc_pool(xa, xb, bt, linW, linb):
    return pl.pallas_call(
        _k5_body,
        grid=(10,),
        in_specs=[
            pl.BlockSpec((_B, H), lambda i: (i, 0)),
            pl.BlockSpec((_B, H), lambda i: (i, 0)),
            pl.BlockSpec((_B, 1), lambda i: (i, 0)),
            pl.BlockSpec((2 * H, OUT), lambda i: (0, 0)),
            pl.BlockSpec((1, OUT), lambda i: (0, 0)),
        ],
        out_specs=pl.BlockSpec((G, OUT), lambda i: (0, 0)),
        out_shape=jax.ShapeDtypeStruct((G, OUT), _f32),
        scratch_shapes=[
            pltpu.VMEM((G, H), _f32),
            pltpu.VMEM((G, H), _f32),
            pltpu.VMEM((G, 8), _f32),
        ],
    )(xa, xb, bt, linW, linb)


def _pad1(a, length, value):
    return jnp.concatenate(
        [a, jnp.full((length - a.shape[0],), value, a.dtype)])


def kernel(x, edge_index, cycle_index, batch, atom_tables, W0, b0, g0, be0,
           W1, b1, g1, be1, W2, b2, g2, be2, Wc, bc, gc, bec, lin_W, lin_b):
    i32 = jnp.int32
    idx_emb = (x.astype(i32).T + 119 * jnp.arange(9, dtype=i32)[:, None]).reshape(-1)
    node_rep = jnp.tile(jnp.arange(N, dtype=i32), 9)
    ep = _pad1(idx_emb, EPM, 0)
    emb_src2 = jnp.stack([2 * ep, 2 * ep + 1]).reshape(32 * SUPM, NB, CHM)
    emb_dst = _pad1(node_rep, EPM, N).reshape(16 * SUPM, NB, CHM)

    def edge_prep(ei):
        sp = _pad1(ei[0].astype(i32), EP, 0)
        src2 = jnp.stack([2 * sp, 2 * sp + 1]).reshape(32 * SUPE, NB, CH)
        dp = _pad1(ei[1].astype(i32), EP, N)
        dstp = dp.reshape(16 * SUPE, NB, CH)
        dstd = dp.reshape(16 * SUPD, 8, DCH)
        return src2, dstp, dstd

    src2_e, dstp_e, dstd_e = edge_prep(edge_index)
    src2_c, dstp_c, dstd_c = edge_prep(cycle_index)
    tcol = atom_tables.reshape(2 * TROWS, 128)
    tcol = jnp.concatenate([tcol, jnp.zeros((TPAD - 2 * TROWS, 128), _f32)])

    x0c = _sc_emb(tcol, emb_src2, emb_dst)
    degc = _sc_deg(dstd_e, dstd_c)
    x0l, x0r = x0c[0:N], x0c[NROWS:NROWS + N]
    hist_e = degc[0:N, 0:1]
    hist_c = degc[NROWS:NROWS + N, 0:1]

    h0, dinv_e, dinv_c, hws0, hwsc = _tc_prep(x0l, x0r, hist_e, hist_c, W0, Wc)

    def agg(hws3, src2, dstp):
        a = _sc_agg(hws3.reshape(2 * N, 128), src2, dstp)
        return a[0:N], a[NROWS:NROWS + N]

    b0r, g0r, be0r = b0.reshape(1, H), g0.reshape(1, H), be0.reshape(1, H)
    b1r, g1r, be1r = b1.reshape(1, H), g1.reshape(1, H), be1.reshape(1, H)
    b2r, g2r, be2r = b2.reshape(1, H), g2.reshape(1, H), be2.reshape(1, H)
    bcr, gcr, becr = bc.reshape(1, H), gc.reshape(1, H), bec.reshape(1, H)

    a0l, a0r = agg(hws0, src2_e, dstp_e)
    y0, s0, q0 = _tc_stats(a0l, a0r, hws0, dinv_e, b0r)
    h1, hws1 = _tc_norm_next(y0, s0, q0, g0r, be0r, h0, W1, dinv_e)
    a1l, a1r = agg(hws1, src2_e, dstp_e)
    y1, s1, q1 = _tc_stats(a1l, a1r, hws1, dinv_e, b1r)
    h2, hws2 = _tc_norm_next(y1, s1, q1, g1r, be1r, h1, W2, dinv_e)
    a2l, a2r = agg(hws2, src2_e, dstp_e)
    y2, s2, q2 = _tc_stats(a2l, a2r, hws2, dinv_e, b2r)
    x_out_a = _tc_norm_final(y2, s2, q2, g2r, be2r, h2)
    acl, acr = agg(hwsc, src2_c, dstp_c)
    yc, scs, qc = _tc_stats(acl, acr, hwsc, dinv_c, bcr)
    x_out_b = _tc_norm_final(yc, scs, qc, gcr, becr, h0)

    bt = batch.astype(i32).reshape(N, 1)
    return _tc_pool(x_out_a, x_out_b, bt, lin_W, lin_b.reshape(1, OUT))


# speedup vs baseline: 1.6874x; 1.0290x over previous
import functools

import jax
import jax.numpy as jnp
from jax import lax
from jax.experimental import pallas as pl
from jax.experimental.pallas import tpu as pltpu
from jax.experimental.pallas import tpu_sc as plsc

N = 10000
H = 256
G = 256
OUT = 128
NROWS = 10240
CH = 128
NB = 2
SUPE = 40
EP = 16 * SUPE * NB * CH
CHM = 64
SUPM = 48
EPM = 16 * SUPM * NB * CHM
TPAD = 2144
SUPD = 10
DCH = 128
TROWS = 9 * 119

_f32 = jnp.float32
_mesh = plsc.VectorSubcoreMesh(core_axis_name="c", subcore_axis_name="s")
_sc_params = pltpu.CompilerParams(use_tc_tiling_on_sc=False)


def _fill2d(ref, nrows, value):
    def body(r, carry):
        ref[r, pl.ds(0, 16)] = jnp.full((16,), value, _f32)
        return carry
    lax.fori_loop(0, nrows, body, 0)


def _zero_buf(ref):
    def body(r, carry):
        for c in range(8):
            ref[r, pl.ds(c * 16, 16)] = jnp.zeros((16,), _f32)
        return carry
    lax.fori_loop(0, ref.shape[0], body, 0)


def _gather_scatter_loop(table, srch, dsth, idxb, dstb, bufs, acc,
                         semg, sems, cid, sid, nsup):
    wrow = (cid * 16 + sid) * nsup
    drow = sid * nsup

    def super_body(s, carry):
        pltpu.sync_copy(srch.at[wrow + s], idxb)
        gs = [pltpu.async_copy(table.at[idxb.at[b]], bufs.at[b], semg)
              for b in range(NB)]
        pltpu.sync_copy(dsth.at[drow + s], dstb)
        ss = []
        for b in range(NB):
            gs[b].wait()
            ss.append(pltpu.async_copy(bufs.at[b], acc.at[dstb.at[b]], sems,
                                       add=True))
        for h in ss:
            h.wait()
        return carry
    lax.fori_loop(0, nsup, super_body, 0)


def _zero_acc(bufs, acc, sid):
    _zero_buf(bufs.at[0])
    for j in range(5):
        pltpu.sync_copy(bufs.at[0], acc.at[pl.ds(sid * 640 + j * CH, CH)])


def _writeout(acc, out, cid, sid):
    for j in range(5):
        pltpu.sync_copy(acc.at[pl.ds(sid * 640 + j * CH, CH)],
                        out.at[pl.ds(cid * NROWS + sid * 640 + j * CH, CH)])


@functools.partial(
    pl.kernel,
    out_type=(jax.ShapeDtypeStruct((2 * NROWS, 128), _f32),
              jax.ShapeDtypeStruct((2 * NROWS, 16), _f32)),
    mesh=_mesh,
    compiler_params=_sc_params,
    scratch_types=[
        pltpu.VMEM((NB, CHM), jnp.int32),
        pltpu.VMEM((NB, CHM), jnp.int32),
        pltpu.VMEM((NB, CHM, 128), _f32),
        pltpu.VMEM((8, DCH), jnp.int32),
        pltpu.VMEM((DCH, 16), _f32),
        pltpu.VMEM((64, 16), _f32),
        pltpu.VMEM_SHARED((NROWS, 128), _f32),
        pltpu.VMEM_SHARED((TPAD, 128), _f32),
        pltpu.VMEM_SHARED((NROWS, 16), _f32),
        pltpu.SemaphoreType.DMA,
        pltpu.SemaphoreType.DMA,
        pltpu.SemaphoreType.DMA,
    ],
)
def _sc_emb(tcol, esrc2, edst, dste, dstc, x0_out, deg_out,
            idxb, dstb, bufs, dstb8, ones16, z16, acc, tsh, accdeg,
            semg, sems, semd):
    cid = lax.axis_index("c")
    sid = lax.axis_index("s")
    rows = TPAD // 16
    pltpu.sync_copy(tcol.at[pl.ds(sid * rows, rows)],
                    tsh.at[pl.ds(sid * rows, rows)])
    _zero_buf(bufs.at[0])
    _fill2d(z16, 64, 0.0)
    _fill2d(ones16, DCH, 1.0)
    for j in range(10):
        pltpu.sync_copy(bufs.at[0], acc.at[pl.ds(sid * 640 + j * CHM, CHM)])
        pltpu.sync_copy(z16, accdeg.at[pl.ds(sid * 640 + j * 64, 64)])
    plsc.subcore_barrier()
    _gather_scatter_loop(tsh, esrc2, edst, idxb, dstb, bufs, acc,
                         semg, sems, cid, sid, SUPM)

    def deg_super(s, carry):
        @pl.when(cid == 0)
        def _():
            pltpu.sync_copy(dste.at[sid * SUPD + s], dstb8)

        @pl.when(cid == 1)
        def _():
            pltpu.sync_copy(dstc.at[sid * SUPD + s], dstb8)

        hs = [pltpu.async_copy(ones16, accdeg.at[dstb8.at[b]], semd, add=True)
              for b in range(8)]
        for h in hs:
            h.wait()
        return carry
    lax.fori_loop(0, SUPD, deg_super, 0)

    plsc.subcore_barrier()
    _writeout(acc, x0_out, cid, sid)
    pltpu.sync_copy(accdeg.at[pl.ds(sid * 640, 640)],
                    deg_out.at[pl.ds(cid * NROWS + sid * 640, 640)])


@functools.partial(
    pl.kernel,
    out_type=jax.ShapeDtypeStruct((2 * NROWS, 128), _f32),
    mesh=_mesh,
    compiler_params=_sc_params,
    scratch_types=[
        pltpu.VMEM((NB, CH), jnp.int32),
        pltpu.VMEM((NB, CH), jnp.int32),
        pltpu.VMEM((NB, CH, 128), _f32),
        pltpu.VMEM_SHARED((NROWS, 128), _f32),
        pltpu.SemaphoreType.DMA,
        pltpu.SemaphoreType.DMA,
    ],
)
def _sc_agg(table, src2, dstp, out, idxb, dstb, bufs, acc, semg, sems):
    cid = lax.axis_index("c")
    sid = lax.axis_index("s")
    _zero_acc(bufs, acc, sid)
    plsc.subcore_barrier()
    _gather_scatter_loop(table, src2, dstp, idxb, dstb, bufs, acc,
                         semg, sems, cid, sid, SUPE)
    plsc.subcore_barrier()
    _writeout(acc, out, cid, sid)


_B = 1000


def _k1_body(x0l, x0r, he, hc, W0, Wc, h0_o, de_o, dc_o, hws0_o, hwsc_o):
    h = jnp.concatenate([x0l[...], x0r[...]], axis=1)
    h0_o[...] = h
    de = lax.rsqrt(he[...] + 1.0)
    dc = lax.rsqrt(hc[...] + 1.0)
    de_o[...] = de
    dc_o[...] = dc
    hws0_o[...] = (jnp.dot(h, W0[...], preferred_element_type=_f32) * de
                   ).reshape(_B, 2, 128)
    hwsc_o[...] = (jnp.dot(h, Wc[...], preferred_element_type=_f32) * dc
                   ).reshape(_B, 2, 128)


def _tc_prep(x0l, x0r, hist_e, hist_c, W0, Wc):
    return pl.pallas_call(
        _k1_body,
        grid=(10,),
        in_specs=[
            pl.BlockSpec((_B, 128), lambda i: (i, 0)),
            pl.BlockSpec((_B, 128), lambda i: (i, 0)),
            pl.BlockSpec((_B, 1), lambda i: (i, 0)),
            pl.BlockSpec((_B, 1), lambda i: (i, 0)),
            pl.BlockSpec((H, H), lambda i: (0, 0)),
            pl.BlockSpec((H, H), lambda i: (0, 0)),
        ],
        out_specs=[
            pl.BlockSpec((_B, H), lambda i: (i, 0)),
            pl.BlockSpec((_B, 1), lambda i: (i, 0)),
            pl.BlockSpec((_B, 1), lambda i: (i, 0)),
            pl.BlockSpec((_B, 2, 128), lambda i: (i, 0, 0)),
            pl.BlockSpec((_B, 2, 128), lambda i: (i, 0, 0)),
        ],
        out_shape=[
            jax.ShapeDtypeStruct((N, H), _f32),
            jax.ShapeDtypeStruct((N, 1), _f32),
            jax.ShapeDtypeStruct((N, 1), _f32),
            jax.ShapeDtypeStruct((N, 2, 128), _f32),
            jax.ShapeDtypeStruct((N, 2, 128), _f32),
        ],
    )(x0l, x0r, hist_e, hist_c, W0, Wc)


def _k2_body(al, ar, hws3, dv, b, y_o, s_o, q_o):
    i = pl.program_id(0)
    agg = jnp.concatenate([al[...], ar[...]], axis=1)
    hws = hws3[...].reshape(_B, H)
    y = dv[...] * (agg + hws) + b[...]
    y_o[...] = y

    @pl.when(i == 0)
    def _():
        s_o[...] = jnp.zeros_like(s_o)
        q_o[...] = jnp.zeros_like(q_o)

    s_o[...] += jnp.sum(y, axis=0, keepdims=True)
    q_o[...] += jnp.sum(y * y, axis=0, keepdims=True)


def _tc_stats(al, ar, hws3, dv, b):
    return pl.pallas_call(
        _k2_body,
        grid=(10,),
        in_specs=[
            pl.BlockSpec((_B, 128), lambda i: (i, 0)),
            pl.BlockSpec((_B, 128), lambda i: (i, 0)),
            pl.BlockSpec((_B, 2, 128), lambda i: (i, 0, 0)),
            pl.BlockSpec((_B, 1), lambda i: (i, 0)),
            pl.BlockSpec((1, H), lambda i: (0, 0)),
        ],
        out_specs=[
            pl.BlockSpec((_B, H), lambda i: (i, 0)),
            pl.BlockSpec((1, H), lambda i: (0, 0)),
            pl.BlockSpec((1, H), lambda i: (0, 0)),
        ],
        out_shape=[
            jax.ShapeDtypeStruct((N, H), _f32),
            jax.ShapeDtypeStruct((1, H), _f32),
            jax.ShapeDtypeStruct((1, H), _f32),
        ],
    )(al, ar, hws3, dv, b)


def _k3_body(y, s, q, g, be, hp, Wn, dv, hn_o, hws_o):
    mu = s[...] / N
    var = q[...] / N - mu * mu
    sc = g[...] * lax.rsqrt(var + 1e-5)
    sh = be[...] - mu * sc
    hn = jnp.maximum(y[...] * sc + sh, 0.0) + hp[...]
    hn_o[...] = hn
    hws_o[...] = (jnp.dot(hn, Wn[...], preferred_element_type=_f32) * dv[...]
                  ).reshape(_B, 2, 128)


def _tc_norm_next(y, s, q, g, be, hp, Wn, dv):
    return pl.pallas_call(
        _k3_body,
        grid=(10,),
        in_specs=[
            pl.BlockSpec((_B, H), lambda i: (i, 0)),
            pl.BlockSpec((1, H), lambda i: (0, 0)),
            pl.BlockSpec((1, H), lambda i: (0, 0)),
            pl.BlockSpec((1, H), lambda i: (0, 0)),
            pl.BlockSpec((1, H), lambda i: (0, 0)),
            pl.BlockSpec((_B, H), lambda i: (i, 0)),
            pl.BlockSpec((H, H), lambda i: (0, 0)),
            pl.BlockSpec((_B, 1), lambda i: (i, 0)),
        ],
        out_specs=[
            pl.BlockSpec((_B, H), lambda i: (i, 0)),
            pl.BlockSpec((_B, 2, 128), lambda i: (i, 0, 0)),
        ],
        out_shape=[
            jax.ShapeDtypeStruct((N, H), _f32),
            jax.ShapeDtypeStruct((N, 2, 128), _f32),
        ],
    )(y, s, q, g, be, hp, Wn, dv)


def _k4_body(y, s, q, g, be, hp, x_o):
    mu = s[...] / N
    var = q[...] / N - mu * mu
    sc = g[...] * lax.rsqrt(var + 1e-5)
    sh = be[...] - mu * sc
    x_o[...] = jnp.maximum(y[...] * sc + sh, 0.0) + hp[...]


def _tc_norm_final(y, s, q, g, be, hp):
    return pl.pallas_call(
        _k4_body,
        grid=(10,),
        in_specs=[
            pl.BlockSpec((_B, H), lambda i: (i, 0)),
            pl.BlockSpec((1, H), lambda i: (0, 0)),
            pl.BlockSpec((1, H), lambda i: (0, 0)),
            pl.BlockSpec((1, H), lambda i: (0, 0)),
            pl.BlockSpec((1, H), lambda i: (0, 0)),
            pl.BlockSpec((_B, H), lambda i: (i, 0)),
        ],
        out_specs=pl.BlockSpec((_B, H), lambda i: (i, 0)),
        out_shape=jax.ShapeDtypeStruct((N, H), _f32),
    )(y, s, q, g, be, hp)


def _k5_body(xa, xb, bt, linW, linb, out_o, Pa, Pb, C8):
    i = pl.program_id(0)

    @pl.when(i == 0)
    def _():
        Pa[...] = jnp.zeros_like(Pa)
        Pb[...] = jnp.zeros_like(Pb)
        C8[...] = jnp.zeros_like(C8)

    oh = (bt[...] == lax.broadcasted_iota(jnp.int32, (1, G), 1)).astype(_f32)
    dims = (((0,), (0,)), ((), ()))
    Pa[...] += lax.dot_general(oh, xa[...], dims, preferred_element_type=_f32)
    Pb[...] += lax.dot_general(oh, xb[...], dims, preferred_element_type=_f32)
    C8[...] += lax.dot_general(oh, jnp.ones((_B, 8), _f32), dims,
                               preferred_element_type=_f32)

    @pl.when(i == 9)
    def _():
        cnt = jnp.maximum(C8[...][:, 0:1], 1.0)
        z = jnp.concatenate([Pa[...] / cnt, Pb[...] / cnt], axis=1)
        out_o[...] = jnp.dot(z, linW[...], preferred_element_type=_f32) + linb[...]


def _tc_pool(xa, xb, bt, linW, linb):
    return pl.pallas_call(
        _k5_body,
        grid=(10,),
        in_specs=[
            pl.BlockSpec((_B, H), lambda i: (i, 0)),
            pl.BlockSpec((_B, H), lambda i: (i, 0)),
            pl.BlockSpec((_B, 1), lambda i: (i, 0)),
            pl.BlockSpec((2 * H, OUT), lambda i: (0, 0)),
            pl.BlockSpec((1, OUT), lambda i: (0, 0)),
        ],
        out_specs=pl.BlockSpec((G, OUT), lambda i: (0, 0)),
        out_shape=jax.ShapeDtypeStruct((G, OUT), _f32),
        scratch_shapes=[
            pltpu.VMEM((G, H), _f32),
            pltpu.VMEM((G, H), _f32),
            pltpu.VMEM((G, 8), _f32),
        ],
    )(xa, xb, bt, linW, linb)


def _pad1(a, length, value):
    return jnp.concatenate(
        [a, jnp.full((length - a.shape[0],), value, a.dtype)])


def kernel(x, edge_index, cycle_index, batch, atom_tables, W0, b0, g0, be0,
           W1, b1, g1, be1, W2, b2, g2, be2, Wc, bc, gc, bec, lin_W, lin_b):
    i32 = jnp.int32
    idx_emb = (x.astype(i32).T + 119 * jnp.arange(9, dtype=i32)[:, None]).reshape(-1)
    node_rep = jnp.tile(jnp.arange(N, dtype=i32), 9)
    ep = _pad1(idx_emb, EPM, 0)
    emb_src2 = jnp.stack([2 * ep, 2 * ep + 1]).reshape(32 * SUPM, NB, CHM)
    emb_dst = _pad1(node_rep, EPM, N).reshape(16 * SUPM, NB, CHM)

    def edge_prep(ei):
        sp = _pad1(ei[0].astype(i32), EP, 0)
        src2 = jnp.stack([2 * sp, 2 * sp + 1]).reshape(32 * SUPE, NB, CH)
        dp = _pad1(ei[1].astype(i32), EP, N)
        dstp = dp.reshape(16 * SUPE, NB, CH)
        dstd = dp.reshape(16 * SUPD, 8, DCH)
        return src2, dstp, dstd

    src2_e, dstp_e, dstd_e = edge_prep(edge_index)
    src2_c, dstp_c, dstd_c = edge_prep(cycle_index)
    tcol = atom_tables.reshape(2 * TROWS, 128)
    tcol = jnp.concatenate([tcol, jnp.zeros((TPAD - 2 * TROWS, 128), _f32)])

    x0c, degc = _sc_emb(tcol, emb_src2, emb_dst, dstd_e, dstd_c)
    x0l, x0r = x0c[0:N], x0c[NROWS:NROWS + N]
    hist_e = degc[0:N, 0:1]
    hist_c = degc[NROWS:NROWS + N, 0:1]

    h0, dinv_e, dinv_c, hws0, hwsc = _tc_prep(x0l, x0r, hist_e, hist_c, W0, Wc)

    def agg(hws3, src2, dstp):
        a = _sc_agg(hws3.reshape(2 * N, 128), src2, dstp)
        return a[0:N], a[NROWS:NROWS + N]

    b0r, g0r, be0r = b0.reshape(1, H), g0.reshape(1, H), be0.reshape(1, H)
    b1r, g1r, be1r = b1.reshape(1, H), g1.reshape(1, H), be1.reshape(1, H)
    b2r, g2r, be2r = b2.reshape(1, H), g2.reshape(1, H), be2.reshape(1, H)
    bcr, gcr, becr = bc.reshape(1, H), gc.reshape(1, H), bec.reshape(1, H)

    a0l, a0r = agg(hws0, src2_e, dstp_e)
    y0, s0, q0 = _tc_stats(a0l, a0r, hws0, dinv_e, b0r)
    h1, hws1 = _tc_norm_next(y0, s0, q0, g0r, be0r, h0, W1, dinv_e)
    a1l, a1r = agg(hws1, src2_e, dstp_e)
    y1, s1, q1 = _tc_stats(a1l, a1r, hws1, dinv_e, b1r)
    h2, hws2 = _tc_norm_next(y1, s1, q1, g1r, be1r, h1, W2, dinv_e)
    a2l, a2r = agg(hws2, src2_e, dstp_e)
    y2, s2, q2 = _tc_stats(a2l, a2r, hws2, dinv_e, b2r)
    x_out_a = _tc_norm_final(y2, s2, q2, g2r, be2r, h2)
    acl, acr = agg(hwsc, src2_c, dstp_c)
    yc, scs, qc = _tc_stats(acl, acr, hwsc, dinv_c, bcr)
    x_out_b = _tc_norm_final(yc, scs, qc, gcr, becr, h0)

    bt = batch.astype(i32).reshape(N, 1)
    return _tc_pool(x_out_a, x_out_b, bt, lin_W, lin_b.reshape(1, OUT))
